# baseline (speedup 1.0000x reference)
import functools

import jax
import jax.numpy as jnp
from jax import lax
from jax.experimental import pallas as pl
from jax.experimental.pallas import tpu as pltpu
from jax.experimental.pallas import tpu_sc as plsc

B, N, F = 64, 8192, 128
L = 16
NT = N // (L * L)
K_HARD, K_CONF = 10, 5
PAD = 16
ROWS_PER_TILE = 2


def _key_hard(s):
    return jnp.abs(s - 0.5)


def _key_id(s):
    return s


def _phase1(score_v, iota, sh_val, sh_u, sn_val, sn_u, sa_val, sa_u,
            l2h, l2n, l2a):
    def merge_min(a, b):
        (va, ua), (vb, ub) = a, b
        m = vb < va
        return jnp.where(m, vb, va), jnp.where(m, ub, ua)

    def merge_max(a, b):
        (va, ua), (vb, ub) = a, b
        m = vb > va
        return jnp.where(m, vb, va), jnp.where(m, ub, ua)

    def tree(leaves, merge):
        while len(leaves) > 1:
            leaves = [merge(leaves[i], leaves[i + 1])
                      for i in range(0, len(leaves), 2)]
        return leaves[0]

    def tt_body(tt, _):
        def body(t2, carry):
            ch, cn, ca = carry
            t = tt * L + t2
            base = t * (L * L)
            ss = [score_v[pl.ds(base + u * L, L)] for u in range(L)]
            us = [jnp.full((L,), u, jnp.int32) for u in range(L)]
            run_h, run_hu = tree([(_key_hard(s), u) for s, u in zip(ss, us)],
                                 merge_min)
            run_n, run_nu = tree(list(zip(ss, us)), merge_min)
            run_a, run_au = tree(list(zip(ss, us)), merge_max)
            sh_val[t] = run_h
            sh_u[t] = run_hu
            sn_val[t] = run_n
            sn_u[t] = run_nu
            sa_val[t] = run_a
            sa_u[t] = run_au
            lm = iota == t2
            ch = jnp.where(lm, jnp.min(run_h), ch)
            cn = jnp.where(lm, jnp.min(run_n), cn)
            ca = jnp.where(lm, jnp.max(run_a), ca)
            return ch, cn, ca

        z = jnp.zeros((L,), jnp.float32)
        ch, cn, ca = lax.fori_loop(0, L, body, (z, z, z))
        l2h[tt] = ch
        l2n[tt] = cn
        l2a[tt] = ca
        return 0

    lax.fori_loop(0, NT // L, tt_body, 0)


def _row_topk(score_v, sval, su, l2, excl, iota, is_min, key_fn, k):
    sentinel = jnp.int32(1 << 30)
    bad = jnp.float32(jnp.inf if is_min else -jnp.inf)
    one = jnp.int32(1)

    def pass_body(p, sel_vec):
        v0 = l2[0]
        v1 = l2[1]
        m01 = (v1 < v0) if is_min else (v1 > v0)
        vbest = jnp.where(m01, v1, v0)
        tbase = jnp.where(m01, L, 0)
        mval = jnp.min(vbest) if is_min else jnp.max(vbest)
        tcand = jnp.where(vbest == mval, tbase + iota, sentinel)
        t_best = jnp.min(tcand)
        sv = sval[t_best]
        uu = su[t_best]
        icand = jnp.where(sv == mval, t_best * (L * L) + uu * L + iota,
                          sentinel)
        g = jnp.min(icand)
        sel_vec = jnp.where(iota == p, g, sel_vec)
        l_sel = lax.bitwise_and(g, L - 1)
        u_sel = lax.bitwise_and(lax.shift_right_logical(g, 4), L - 1)
        lanem = iota == l_sel
        erow = excl[t_best]
        e_new = jnp.bitwise_or(jnp.max(jnp.where(lanem, erow, 0)),
                               lax.shift_left(one, u_sel))
        excl[t_best] = jnp.where(lanem, e_new, erow)
        bidx = t_best * (L * L) + iota * L + l_sel
        key = key_fn(plsc.load_gather(score_v, [bidx]))
        em = lax.bitwise_and(lax.shift_right_logical(e_new, iota), one) == one
        keym = jnp.where(em, bad, key)
        vnew = jnp.min(keym) if is_min else jnp.max(keym)
        ufirst = jnp.min(jnp.where(keym == vnew, iota, jnp.int32(L)))
        newsv = jnp.where(lanem, vnew, sv)
        sval[t_best] = newsv
        su[t_best] = jnp.where(lanem, ufirst, uu)
        l2row = lax.shift_right_logical(t_best, 4)
        l2new = jnp.min(newsv) if is_min else jnp.max(newsv)
        l2lane = lax.bitwise_and(t_best, L - 1)
        l2[l2row] = jnp.where(iota == l2lane, l2new, l2[l2row])
        return sel_vec

    sel_vec = lax.fori_loop(0, k, pass_body, jnp.zeros((L,), jnp.int32))
    plsc.store_scatter(
        excl,
        [lax.shift_right_logical(sel_vec, 8), lax.bitwise_and(sel_vec, L - 1)],
        jnp.zeros((L,), jnp.int32))
    return sel_vec


def _body(feat_hbm, score_hbm,
          o_fp, o_vp, o_ip,
          score_v, sh_val, sh_u, sn_val, sn_u, sa_val, sa_u,
          l2h, l2n, l2a, exh, exn, exa,
          idx_v, gidx_v, vals_v, rows_v, sem):
    cid = lax.axis_index("c")
    sid = lax.axis_index("s")
    wid = sid * 2 + cid
    iota = lax.iota(jnp.int32, L)
    zf = jnp.zeros((L,), jnp.float32)
    zi = jnp.zeros((L,), jnp.int32)
    for q in range(2 * F // L):
        vals_v[pl.ds(q * L, L)] = zf
        idx_v[pl.ds(q * L, L)] = zi

    def zero_body(t, _):
        exh[t] = zi
        exn[t] = zi
        exa[t] = zi
        return 0

    lax.fori_loop(0, NT, zero_body, 0)

    def row_body(r, _):
        b = wid * ROWS_PER_TILE + r
        with jax.named_scope("dma_in"):
            pltpu.sync_copy(score_hbm.at[b], score_v)
        with jax.named_scope("phase1"):
            _phase1(score_v, iota, sh_val, sh_u, sn_val, sn_u, sa_val, sa_u,
                    l2h, l2n, l2a)
        specs = (
            (True, _key_id, K_CONF, sn_val, sn_u, l2n, exn, 0),
            (False, _key_id, K_CONF, sa_val, sa_u, l2a, exa, 1),
            (True, _key_hard, K_HARD, sh_val, sh_u, l2h, exh, 2),
        )
        with jax.named_scope("select"):
            for is_min, key_fn, k, sval, su, l2, exc, slot in specs:
                sel = _row_topk(score_v, sval, su, l2, exc, iota, is_min,
                                key_fn, k)
                idx_v[pl.ds(r * F + slot * L, L)] = sel
                vals_v[pl.ds(r * F + slot * L, L)] = plsc.load_gather(
                    score_v, [sel])
                gidx_v[pl.ds(slot * L, L)] = sel + b * N
        with jax.named_scope("dma_out"):
            pltpu.async_copy(feat_hbm.at[gidx_v], rows_v, sem).wait()
            pltpu.sync_copy(rows_v, o_fp.at[b])
            pltpu.sync_copy(vals_v.at[pl.ds(r * F, F)], o_vp.at[b])
            pltpu.sync_copy(idx_v.at[pl.ds(r * F, F)], o_ip.at[b])
        return 0

    lax.fori_loop(0, ROWS_PER_TILE, row_body, 0)


_mesh = plsc.VectorSubcoreMesh(core_axis_name="c", subcore_axis_name="s")

_sc_call = pl.kernel(
    _body,
    out_type=[
        jax.ShapeDtypeStruct((B, 3 * L, F), jnp.float32),
        jax.ShapeDtypeStruct((B, F), jnp.float32),
        jax.ShapeDtypeStruct((B, F), jnp.int32),
    ],
    mesh=_mesh,
    compiler_params=pltpu.CompilerParams(needs_layout_passes=False),
    scratch_types=[
        pltpu.VMEM((N,), jnp.float32),
        pltpu.VMEM((NT, L), jnp.float32),
        pltpu.VMEM((NT, L), jnp.int32),
        pltpu.VMEM((NT, L), jnp.float32),
        pltpu.VMEM((NT, L), jnp.int32),
        pltpu.VMEM((NT, L), jnp.float32),
        pltpu.VMEM((NT, L), jnp.int32),
        pltpu.VMEM((NT // L, L), jnp.float32),
        pltpu.VMEM((NT // L, L), jnp.float32),
        pltpu.VMEM((NT // L, L), jnp.float32),
        pltpu.VMEM((NT, L), jnp.int32),
        pltpu.VMEM((NT, L), jnp.int32),
        pltpu.VMEM((NT, L), jnp.int32),
        pltpu.VMEM((2 * F,), jnp.int32),
        pltpu.VMEM((3 * L,), jnp.int32),
        pltpu.VMEM((2 * F,), jnp.float32),
        pltpu.VMEM((3 * L, F), jnp.float32),
        pltpu.SemaphoreType.DMA,
    ],
)


def _tc_post_body(fp_ref, vp_ref, ip_ref,
                  o_fn, o_sn, o_in, o_fa, o_sa, o_ia, o_fh, o_sh, o_ih):
    fp = fp_ref[...]
    for k in range(K_CONF):
        o_fn[k] = fp[:, k, :]
        o_fa[k] = fp[:, L + k, :]
    for k in range(K_HARD):
        o_fh[k] = fp[:, 2 * L + k, :]
    vpt = jnp.swapaxes(vp_ref[...], 0, 1)
    o_sn[...] = vpt[0:K_CONF, :]
    o_sa[...] = vpt[L:L + K_CONF, :]
    o_sh[...] = vpt[2 * L:2 * L + K_HARD, :]
    ipt = jnp.swapaxes(ip_ref[...], 0, 1)
    o_in[...] = ipt[0:K_CONF, :]
    o_ia[...] = ipt[L:L + K_CONF, :]
    o_ih[...] = ipt[2 * L:2 * L + K_HARD, :]


_tc_post = pl.pallas_call(
    _tc_post_body,
    out_shape=[
        jax.ShapeDtypeStruct((K_CONF, B, F), jnp.float32),
        jax.ShapeDtypeStruct((K_CONF, B), jnp.float32),
        jax.ShapeDtypeStruct((K_CONF, B), jnp.int32),
        jax.ShapeDtypeStruct((K_CONF, B, F), jnp.float32),
        jax.ShapeDtypeStruct((K_CONF, B), jnp.float32),
        jax.ShapeDtypeStruct((K_CONF, B), jnp.int32),
        jax.ShapeDtypeStruct((K_HARD, B, F), jnp.float32),
        jax.ShapeDtypeStruct((K_HARD, B), jnp.float32),
        jax.ShapeDtypeStruct((K_HARD, B), jnp.int32),
    ],
)


@jax.jit
def kernel(feat, score):
    feat_flat = feat.reshape(B * N, F)
    fp, vp, ip = _sc_call(feat_flat, score)
    fn, sn, i_n, fa, sa, i_a, fh, sh, i_h = _tc_post(fp, vp, ip)
    return (jnp.swapaxes(fn, 0, 1), sn.T, i_n.T,
            jnp.swapaxes(fa, 0, 1), sa.T, i_a.T,
            jnp.swapaxes(fh, 0, 1), sh.T, i_h.T)


# speedup vs baseline: 1.0359x; 1.0359x over previous
import functools

import jax
import jax.numpy as jnp
from jax import lax
from jax.experimental import pallas as pl
from jax.experimental.pallas import tpu as pltpu
from jax.experimental.pallas import tpu_sc as plsc

B, N, F = 64, 8192, 128
L = 16
NT = N // (L * L)
K_HARD, K_CONF = 10, 5
PAD = 16
ROWS_PER_TILE = 2


def _key_hard(s):
    return jnp.abs(s - 0.5)


def _key_id(s):
    return s


def _phase1(score_v, roff, iota, sh_val, sh_u, sn_val, sn_u, sa_val, sa_u,
            l2h, l2n, l2a):
    def merge_min(a, b):
        (va, ua), (vb, ub) = a, b
        m = vb < va
        return jnp.where(m, vb, va), jnp.where(m, ub, ua)

    def merge_max(a, b):
        (va, ua), (vb, ub) = a, b
        m = vb > va
        return jnp.where(m, vb, va), jnp.where(m, ub, ua)

    def tree(leaves, merge):
        while len(leaves) > 1:
            leaves = [merge(leaves[i], leaves[i + 1])
                      for i in range(0, len(leaves), 2)]
        return leaves[0]

    def tt_body(tt, _):
        def body(t2, carry):
            ch, cn, ca = carry
            t = tt * L + t2
            base = t * (L * L)
            ss = [score_v[pl.ds(roff + base + u * L, L)] for u in range(L)]
            us = [jnp.full((L,), u, jnp.int32) for u in range(L)]
            run_h, run_hu = tree([(_key_hard(s), u) for s, u in zip(ss, us)],
                                 merge_min)
            run_n, run_nu = tree(list(zip(ss, us)), merge_min)
            run_a, run_au = tree(list(zip(ss, us)), merge_max)
            sh_val[t] = run_h
            sh_u[t] = run_hu
            sn_val[t] = run_n
            sn_u[t] = run_nu
            sa_val[t] = run_a
            sa_u[t] = run_au
            lm = iota == t2
            ch = jnp.where(lm, jnp.min(run_h), ch)
            cn = jnp.where(lm, jnp.min(run_n), cn)
            ca = jnp.where(lm, jnp.max(run_a), ca)
            return ch, cn, ca

        z = jnp.zeros((L,), jnp.float32)
        ch, cn, ca = lax.fori_loop(0, L, body, (z, z, z))
        l2h[tt] = ch
        l2n[tt] = cn
        l2a[tt] = ca
        return 0

    lax.fori_loop(0, NT // L, tt_body, 0)


def _row_topk(score_v, roff, sval, su, l2, excl, iota, is_min, key_fn, k):
    sentinel = jnp.int32(1 << 30)
    bad = jnp.float32(jnp.inf if is_min else -jnp.inf)
    one = jnp.int32(1)

    def pass_body(p, sel_vec):
        v0 = l2[0]
        v1 = l2[1]
        m01 = (v1 < v0) if is_min else (v1 > v0)
        vbest = jnp.where(m01, v1, v0)
        tbase = jnp.where(m01, L, 0)
        mval = jnp.min(vbest) if is_min else jnp.max(vbest)
        tcand = jnp.where(vbest == mval, tbase + iota, sentinel)
        t_best = jnp.min(tcand)
        sv = sval[t_best]
        uu = su[t_best]
        icand = jnp.where(sv == mval, t_best * (L * L) + uu * L + iota,
                          sentinel)
        g = jnp.min(icand)
        sel_vec = jnp.where(iota == p, g, sel_vec)
        l_sel = lax.bitwise_and(g, L - 1)
        u_sel = lax.bitwise_and(lax.shift_right_logical(g, 4), L - 1)
        lanem = iota == l_sel
        erow = excl[t_best]
        e_new = jnp.bitwise_or(jnp.max(jnp.where(lanem, erow, 0)),
                               lax.shift_left(one, u_sel))
        excl[t_best] = jnp.where(lanem, e_new, erow)
        bidx = t_best * (L * L) + iota * L + l_sel
        key = key_fn(plsc.load_gather(score_v, [bidx + roff]))
        em = lax.bitwise_and(lax.shift_right_logical(e_new, iota), one) == one
        keym = jnp.where(em, bad, key)
        vnew = jnp.min(keym) if is_min else jnp.max(keym)
        ufirst = jnp.min(jnp.where(keym == vnew, iota, jnp.int32(L)))
        newsv = jnp.where(lanem, vnew, sv)
        sval[t_best] = newsv
        su[t_best] = jnp.where(lanem, ufirst, uu)
        l2row = lax.shift_right_logical(t_best, 4)
        l2new = jnp.min(newsv) if is_min else jnp.max(newsv)
        l2lane = lax.bitwise_and(t_best, L - 1)
        l2[l2row] = jnp.where(iota == l2lane, l2new, l2[l2row])
        return sel_vec

    sel_vec = lax.fori_loop(0, k, pass_body, jnp.zeros((L,), jnp.int32))
    plsc.store_scatter(
        excl,
        [lax.shift_right_logical(sel_vec, 8), lax.bitwise_and(sel_vec, L - 1)],
        jnp.zeros((L,), jnp.int32))
    return sel_vec


def _body(feat_hbm, score_hbm,
          o_fp, o_vp, o_ip,
          score_v, sh_val, sh_u, sn_val, sn_u, sa_val, sa_u,
          l2h, l2n, l2a, exh, exn, exa,
          idx_v, gidx_v, vals_v, rows_v, sem_in, sem_g, sem_out):
    cid = lax.axis_index("c")
    sid = lax.axis_index("s")
    wid = sid * 2 + cid
    iota = lax.iota(jnp.int32, L)
    zf = jnp.zeros((L,), jnp.float32)
    zi = jnp.zeros((L,), jnp.int32)
    for q in range(2 * F // L):
        vals_v[pl.ds(q * L, L)] = zf
        idx_v[pl.ds(q * L, L)] = zi

    def zero_body(t, _):
        exh[t] = zi
        exn[t] = zi
        exa[t] = zi
        return 0

    lax.fori_loop(0, NT, zero_body, 0)

    def score_copy(r, start):
        b = wid * ROWS_PER_TILE + r
        h = pltpu.make_async_copy(score_hbm.at[b],
                                  score_v.at[pl.ds((r & 1) * N, N)], sem_in)
        if start:
            h.start()
        else:
            h.wait()

    score_copy(0, True)

    def row_body(r, _):
        rb = r & 1
        roff = rb * N
        b = wid * ROWS_PER_TILE + r
        with jax.named_scope("dma_in"):
            score_copy(r, False)

            @pl.when(r + 1 < ROWS_PER_TILE)
            def _():
                score_copy(r + 1, True)

        with jax.named_scope("phase1"):
            _phase1(score_v, roff, iota, sh_val, sh_u, sn_val, sn_u,
                    sa_val, sa_u, l2h, l2n, l2a)
        specs = (
            (True, _key_id, K_CONF, sn_val, sn_u, l2n, exn, 0),
            (False, _key_id, K_CONF, sa_val, sa_u, l2a, exa, 1),
            (True, _key_hard, K_HARD, sh_val, sh_u, l2h, exh, 2),
        )
        with jax.named_scope("select"):
            for is_min, key_fn, k, sval, su, l2, exc, slot in specs:
                sel = _row_topk(score_v, roff, sval, su, l2, exc, iota,
                                is_min, key_fn, k)
                idx_v[pl.ds(r * F + slot * L, L)] = sel
                vals_v[pl.ds(r * F + slot * L, L)] = plsc.load_gather(
                    score_v, [sel + roff])
                gidx_v[pl.ds(slot * L, L)] = sel + b * N
        with jax.named_scope("dma_out"):
            gh = pltpu.make_async_copy(
                feat_hbm.at[gidx_v], rows_v.at[pl.ds(rb * 3 * L, 3 * L)],
                sem_g)
            gh.start()
            pltpu.make_async_copy(vals_v.at[pl.ds(r * F, F)], o_vp.at[b],
                                  sem_out).start()
            pltpu.make_async_copy(idx_v.at[pl.ds(r * F, F)], o_ip.at[b],
                                  sem_out).start()
            gh.wait()
            pltpu.make_async_copy(rows_v.at[pl.ds(rb * 3 * L, 3 * L)],
                                  o_fp.at[b], sem_out).start()
        return 0

    lax.fori_loop(0, ROWS_PER_TILE, row_body, 0)
    with jax.named_scope("drain"):
        for r in range(ROWS_PER_TILE):
            b = wid * ROWS_PER_TILE + r
            pltpu.make_async_copy(vals_v.at[pl.ds(r * F, F)], o_vp.at[b],
                                  sem_out).wait()
            pltpu.make_async_copy(idx_v.at[pl.ds(r * F, F)], o_ip.at[b],
                                  sem_out).wait()
            pltpu.make_async_copy(rows_v.at[pl.ds((r & 1) * 3 * L, 3 * L)],
                                  o_fp.at[b], sem_out).wait()


_mesh = plsc.VectorSubcoreMesh(core_axis_name="c", subcore_axis_name="s")

_sc_call = pl.kernel(
    _body,
    out_type=[
        jax.ShapeDtypeStruct((B, 3 * L, F), jnp.float32),
        jax.ShapeDtypeStruct((B, F), jnp.float32),
        jax.ShapeDtypeStruct((B, F), jnp.int32),
    ],
    mesh=_mesh,
    compiler_params=pltpu.CompilerParams(needs_layout_passes=False),
    scratch_types=[
        pltpu.VMEM((2 * N,), jnp.float32),
        pltpu.VMEM((NT, L), jnp.float32),
        pltpu.VMEM((NT, L), jnp.int32),
        pltpu.VMEM((NT, L), jnp.float32),
        pltpu.VMEM((NT, L), jnp.int32),
        pltpu.VMEM((NT, L), jnp.float32),
        pltpu.VMEM((NT, L), jnp.int32),
        pltpu.VMEM((NT // L, L), jnp.float32),
        pltpu.VMEM((NT // L, L), jnp.float32),
        pltpu.VMEM((NT // L, L), jnp.float32),
        pltpu.VMEM((NT, L), jnp.int32),
        pltpu.VMEM((NT, L), jnp.int32),
        pltpu.VMEM((NT, L), jnp.int32),
        pltpu.VMEM((2 * F,), jnp.int32),
        pltpu.VMEM((3 * L,), jnp.int32),
        pltpu.VMEM((2 * F,), jnp.float32),
        pltpu.VMEM((2 * 3 * L, F), jnp.float32),
        pltpu.SemaphoreType.DMA,
        pltpu.SemaphoreType.DMA,
        pltpu.SemaphoreType.DMA,
    ],
)


def _tc_post_body(fp_ref, vp_ref, ip_ref,
                  o_fn, o_sn, o_in, o_fa, o_sa, o_ia, o_fh, o_sh, o_ih):
    fp = fp_ref[...]
    for k in range(K_CONF):
        o_fn[k] = fp[:, k, :]
        o_fa[k] = fp[:, L + k, :]
    for k in range(K_HARD):
        o_fh[k] = fp[:, 2 * L + k, :]
    vpt = jnp.swapaxes(vp_ref[...], 0, 1)
    o_sn[...] = vpt[0:K_CONF, :]
    o_sa[...] = vpt[L:L + K_CONF, :]
    o_sh[...] = vpt[2 * L:2 * L + K_HARD, :]
    ipt = jnp.swapaxes(ip_ref[...], 0, 1)
    o_in[...] = ipt[0:K_CONF, :]
    o_ia[...] = ipt[L:L + K_CONF, :]
    o_ih[...] = ipt[2 * L:2 * L + K_HARD, :]


_tc_post = pl.pallas_call(
    _tc_post_body,
    out_shape=[
        jax.ShapeDtypeStruct((K_CONF, B, F), jnp.float32),
        jax.ShapeDtypeStruct((K_CONF, B), jnp.float32),
        jax.ShapeDtypeStruct((K_CONF, B), jnp.int32),
        jax.ShapeDtypeStruct((K_CONF, B, F), jnp.float32),
        jax.ShapeDtypeStruct((K_CONF, B), jnp.float32),
        jax.ShapeDtypeStruct((K_CONF, B), jnp.int32),
        jax.ShapeDtypeStruct((K_HARD, B, F), jnp.float32),
        jax.ShapeDtypeStruct((K_HARD, B), jnp.float32),
        jax.ShapeDtypeStruct((K_HARD, B), jnp.int32),
    ],
)


@jax.jit
def kernel(feat, score):
    feat_flat = feat.reshape(B * N, F)
    fp, vp, ip = _sc_call(feat_flat, score)
    fn, sn, i_n, fa, sa, i_a, fh, sh, i_h = _tc_post(fp, vp, ip)
    return (jnp.swapaxes(fn, 0, 1), sn.T, i_n.T,
            jnp.swapaxes(fa, 0, 1), sa.T, i_a.T,
            jnp.swapaxes(fh, 0, 1), sh.T, i_h.T)


# speedup vs baseline: 1.1662x; 1.1257x over previous
import functools

import jax
import jax.numpy as jnp
from jax import lax
from jax.experimental import pallas as pl
from jax.experimental.pallas import tpu as pltpu
from jax.experimental.pallas import tpu_sc as plsc

B, N, F = 64, 8192, 128
L = 16
NT = N // (L * L)
K_HARD, K_CONF = 10, 5
PAD = 16
ROWS_PER_TILE = 2


def _key_hard(s):
    return jnp.abs(s - 0.5)


def _key_id(s):
    return s


def _phase1(score_v, roff, iota, sh_val, sh_u, sn_val, sn_u, sa_val, sa_u,
            l2h, l2n, l2a):
    def merge_min(a, b):
        (va, ua), (vb, ub) = a, b
        m = vb < va
        return jnp.where(m, vb, va), jnp.where(m, ub, ua)

    def merge_max(a, b):
        (va, ua), (vb, ub) = a, b
        m = vb > va
        return jnp.where(m, vb, va), jnp.where(m, ub, ua)

    def tree(leaves, merge):
        while len(leaves) > 1:
            leaves = [merge(leaves[i], leaves[i + 1])
                      for i in range(0, len(leaves), 2)]
        return leaves[0]

    def tt_body(tt, _):
        def body(t2, carry):
            ch, cn, ca = carry
            t = tt * L + t2
            base = t * (L * L)
            ss = [score_v[pl.ds(roff + base + u * L, L)] for u in range(L)]
            us = [jnp.full((L,), u, jnp.int32) for u in range(L)]
            run_h, run_hu = tree([(_key_hard(s), u) for s, u in zip(ss, us)],
                                 merge_min)
            run_n, run_nu = tree(list(zip(ss, us)), merge_min)
            run_a, run_au = tree(list(zip(ss, us)), merge_max)
            sh_val[t] = run_h
            sh_u[t] = run_hu
            sn_val[t] = run_n
            sn_u[t] = run_nu
            sa_val[t] = run_a
            sa_u[t] = run_au
            lm = iota == t2
            ch = jnp.where(lm, jnp.min(run_h), ch)
            cn = jnp.where(lm, jnp.min(run_n), cn)
            ca = jnp.where(lm, jnp.max(run_a), ca)
            return ch, cn, ca

        z = jnp.zeros((L,), jnp.float32)
        ch, cn, ca = lax.fori_loop(0, L, body, (z, z, z))
        l2h[tt] = ch
        l2n[tt] = cn
        l2a[tt] = ca
        return 0

    lax.fori_loop(0, NT // L, tt_body, 0)


def _row_topk(score_v, roff, sval, su, l2, excl, iota, is_min, key_fn, k):
    sentinel = jnp.int32(1 << 30)
    bad = jnp.float32(jnp.inf if is_min else -jnp.inf)
    one = jnp.int32(1)

    def pass_body(p, sel_vec):
        v0 = l2[0]
        v1 = l2[1]
        m01 = (v1 < v0) if is_min else (v1 > v0)
        vbest = jnp.where(m01, v1, v0)
        tbase = jnp.where(m01, L, 0)
        mval = jnp.min(vbest) if is_min else jnp.max(vbest)
        tcand = jnp.where(vbest == mval, tbase + iota, sentinel)
        t_best = jnp.min(tcand)
        sv = sval[t_best]
        uu = su[t_best]
        icand = jnp.where(sv == mval, t_best * (L * L) + uu * L + iota,
                          sentinel)
        g = jnp.min(icand)
        sel_vec = jnp.where(iota == p, g, sel_vec)
        l_sel = lax.bitwise_and(g, L - 1)
        u_sel = lax.bitwise_and(lax.shift_right_logical(g, 4), L - 1)
        lanem = iota == l_sel
        erow = excl[t_best]
        e_new = jnp.bitwise_or(jnp.max(jnp.where(lanem, erow, 0)),
                               lax.shift_left(one, u_sel))
        excl[t_best] = jnp.where(lanem, e_new, erow)
        bidx = t_best * (L * L) + iota * L + l_sel
        key = key_fn(plsc.load_gather(score_v, [bidx + roff]))
        em = lax.bitwise_and(lax.shift_right_logical(e_new, iota), one) == one
        keym = jnp.where(em, bad, key)
        vnew = jnp.min(keym) if is_min else jnp.max(keym)
        ufirst = jnp.min(jnp.where(keym == vnew, iota, jnp.int32(L)))
        newsv = jnp.where(lanem, vnew, sv)
        sval[t_best] = newsv
        su[t_best] = jnp.where(lanem, ufirst, uu)
        l2row = lax.shift_right_logical(t_best, 4)
        l2new = jnp.min(newsv) if is_min else jnp.max(newsv)
        l2lane = lax.bitwise_and(t_best, L - 1)
        l2[l2row] = jnp.where(iota == l2lane, l2new, l2[l2row])
        return sel_vec

    sel_vec = lax.fori_loop(0, k, pass_body, jnp.zeros((L,), jnp.int32))
    plsc.store_scatter(
        excl,
        [lax.shift_right_logical(sel_vec, 8), lax.bitwise_and(sel_vec, L - 1)],
        jnp.zeros((L,), jnp.int32))
    return sel_vec


def _body(feat_hbm, score_hbm,
          o_fp, o_vp, o_ip,
          score_v, sh_val, sh_u, sn_val, sn_u, sa_val, sa_u,
          l2h, l2n, l2a, exh, exn, exa,
          idx_v, gidx_v, vals_v, rows_v, sem_in, sem_g, sem_out):
    cid = lax.axis_index("c")
    sid = lax.axis_index("s")
    wid = sid * 2 + cid
    iota = lax.iota(jnp.int32, L)
    zf = jnp.zeros((L,), jnp.float32)
    zi = jnp.zeros((L,), jnp.int32)
    for q in range(2 * F // L):
        vals_v[pl.ds(q * L, L)] = zf
        idx_v[pl.ds(q * L, L)] = zi

    def zero_body(t, _):
        exh[t] = zi
        exn[t] = zi
        exa[t] = zi
        return 0

    lax.fori_loop(0, NT, zero_body, 0)

    def score_copy(r, start):
        b = wid * ROWS_PER_TILE + r
        h = pltpu.make_async_copy(score_hbm.at[b],
                                  score_v.at[pl.ds((r & 1) * N, N)], sem_in)
        if start:
            h.start()
        else:
            h.wait()

    score_copy(0, True)

    def row_body(r, _):
        rb = r & 1
        roff = rb * N
        b = wid * ROWS_PER_TILE + r
        with jax.named_scope("dma_in"):
            score_copy(r, False)

            @pl.when(r + 1 < ROWS_PER_TILE)
            def _():
                score_copy(r + 1, True)

        with jax.named_scope("phase1"):
            _phase1(score_v, roff, iota, sh_val, sh_u, sn_val, sn_u,
                    sa_val, sa_u, l2h, l2n, l2a)
        specs = (
            (True, _key_id, K_CONF, sn_val, sn_u, l2n, exn, 0),
            (False, _key_id, K_CONF, sa_val, sa_u, l2a, exa, 1),
            (True, _key_hard, K_HARD, sh_val, sh_u, l2h, exh, 2),
        )
        with jax.named_scope("select"):
            for is_min, key_fn, k, sval, su, l2, exc, slot in specs:
                sel = _row_topk(score_v, roff, sval, su, l2, exc, iota,
                                is_min, key_fn, k)
                idx_v[pl.ds(r * F + slot * L, L)] = sel
                vals_v[pl.ds(r * F + slot * L, L)] = plsc.load_gather(
                    score_v, [sel + roff])
                gs = pl.ds(rb * 3 * L + slot * L, L)
                gidx_v[gs] = sel + b * N
                pltpu.make_async_copy(feat_hbm.at[gidx_v.at[gs]],
                                      rows_v.at[gs], sem_g).start()
        with jax.named_scope("dma_out"):
            pltpu.make_async_copy(vals_v.at[pl.ds(r * F, F)], o_vp.at[b],
                                  sem_out).start()
            pltpu.make_async_copy(idx_v.at[pl.ds(r * F, F)], o_ip.at[b],
                                  sem_out).start()
        return 0

    lax.fori_loop(0, ROWS_PER_TILE, row_body, 0)
    with jax.named_scope("drain"):
        for r in range(ROWS_PER_TILE):
            for slot in range(3):
                gs = pl.ds((r & 1) * 3 * L + slot * L, L)
                pltpu.make_async_copy(feat_hbm.at[gidx_v.at[gs]],
                                      rows_v.at[gs], sem_g).wait()
        for r in range(ROWS_PER_TILE):
            b = wid * ROWS_PER_TILE + r
            pltpu.make_async_copy(rows_v.at[pl.ds((r & 1) * 3 * L, 3 * L)],
                                  o_fp.at[b], sem_out).start()
        for r in range(ROWS_PER_TILE):
            b = wid * ROWS_PER_TILE + r
            pltpu.make_async_copy(vals_v.at[pl.ds(r * F, F)], o_vp.at[b],
                                  sem_out).wait()
            pltpu.make_async_copy(idx_v.at[pl.ds(r * F, F)], o_ip.at[b],
                                  sem_out).wait()
            pltpu.make_async_copy(rows_v.at[pl.ds((r & 1) * 3 * L, 3 * L)],
                                  o_fp.at[b], sem_out).wait()


_mesh = plsc.VectorSubcoreMesh(core_axis_name="c", subcore_axis_name="s")

_sc_call = pl.kernel(
    _body,
    out_type=[
        jax.ShapeDtypeStruct((B, 3 * L, F), jnp.float32),
        jax.ShapeDtypeStruct((B, F), jnp.float32),
        jax.ShapeDtypeStruct((B, F), jnp.int32),
    ],
    mesh=_mesh,
    compiler_params=pltpu.CompilerParams(needs_layout_passes=False),
    scratch_types=[
        pltpu.VMEM((2 * N,), jnp.float32),
        pltpu.VMEM((NT, L), jnp.float32),
        pltpu.VMEM((NT, L), jnp.int32),
        pltpu.VMEM((NT, L), jnp.float32),
        pltpu.VMEM((NT, L), jnp.int32),
        pltpu.VMEM((NT, L), jnp.float32),
        pltpu.VMEM((NT, L), jnp.int32),
        pltpu.VMEM((NT // L, L), jnp.float32),
        pltpu.VMEM((NT // L, L), jnp.float32),
        pltpu.VMEM((NT // L, L), jnp.float32),
        pltpu.VMEM((NT, L), jnp.int32),
        pltpu.VMEM((NT, L), jnp.int32),
        pltpu.VMEM((NT, L), jnp.int32),
        pltpu.VMEM((2 * F,), jnp.int32),
        pltpu.VMEM((2 * 3 * L,), jnp.int32),
        pltpu.VMEM((2 * F,), jnp.float32),
        pltpu.VMEM((2 * 3 * L, F), jnp.float32),
        pltpu.SemaphoreType.DMA,
        pltpu.SemaphoreType.DMA,
        pltpu.SemaphoreType.DMA,
    ],
)


def _tc_post_body(fp_ref, vp_ref, ip_ref,
                  o_fn, o_sn, o_in, o_fa, o_sa, o_ia, o_fh, o_sh, o_ih):
    fp = fp_ref[...]
    for k in range(K_CONF):
        o_fn[k] = fp[:, k, :]
        o_fa[k] = fp[:, L + k, :]
    for k in range(K_HARD):
        o_fh[k] = fp[:, 2 * L + k, :]
    vpt = jnp.swapaxes(vp_ref[...], 0, 1)
    o_sn[...] = vpt[0:K_CONF, :]
    o_sa[...] = vpt[L:L + K_CONF, :]
    o_sh[...] = vpt[2 * L:2 * L + K_HARD, :]
    ipt = jnp.swapaxes(ip_ref[...], 0, 1)
    o_in[...] = ipt[0:K_CONF, :]
    o_ia[...] = ipt[L:L + K_CONF, :]
    o_ih[...] = ipt[2 * L:2 * L + K_HARD, :]


_tc_post = pl.pallas_call(
    _tc_post_body,
    out_shape=[
        jax.ShapeDtypeStruct((K_CONF, B, F), jnp.float32),
        jax.ShapeDtypeStruct((K_CONF, B), jnp.float32),
        jax.ShapeDtypeStruct((K_CONF, B), jnp.int32),
        jax.ShapeDtypeStruct((K_CONF, B, F), jnp.float32),
        jax.ShapeDtypeStruct((K_CONF, B), jnp.float32),
        jax.ShapeDtypeStruct((K_CONF, B), jnp.int32),
        jax.ShapeDtypeStruct((K_HARD, B, F), jnp.float32),
        jax.ShapeDtypeStruct((K_HARD, B), jnp.float32),
        jax.ShapeDtypeStruct((K_HARD, B), jnp.int32),
    ],
)


@jax.jit
def kernel(feat, score):
    feat_flat = feat.reshape(B * N, F)
    fp, vp, ip = _sc_call(feat_flat, score)
    fn, sn, i_n, fa, sa, i_a, fh, sh, i_h = _tc_post(fp, vp, ip)
    return (jnp.swapaxes(fn, 0, 1), sn.T, i_n.T,
            jnp.swapaxes(fa, 0, 1), sa.T, i_a.T,
            jnp.swapaxes(fh, 0, 1), sh.T, i_h.T)
